# baseline (speedup 1.0000x reference)
import functools

import jax
import jax.numpy as jnp
from jax import lax
from jax.experimental import pallas as pl
from jax.experimental.pallas import tpu as pltpu
from jax.experimental.pallas import tpu_sc as plsc

N, E, D, B = 10000, 160000, 256, 64
L1, L2, L3 = 40000, 80000, 160000

NC, NS = 2, 16
NW = NC * NS
LANES = 16

CA = 80
CB1 = 64
CB2 = 128

_MESH = plsc.VectorSubcoreMesh(
    core_axis_name="c", subcore_axis_name="s", num_cores=NC, num_subcores=NS
)


def _worker_id():
    return lax.axis_index("s") * NC + lax.axis_index("c")


@functools.partial(
    pl.kernel,
    out_type=jax.ShapeDtypeStruct((N, D), jnp.float32),
    mesh=_MESH,
    scratch_types=[
        pltpu.VMEM((CA,), jnp.int32),
        pltpu.VMEM((CA, D), jnp.float32),
        pltpu.VMEM((CA, D), jnp.float32),
        pltpu.SemaphoreType.DMA,
    ],
)
def _tng0_call(node_hbm, gfeat_hbm, nb_hbm, out_hbm, nb_v, g_v, nf_v, sem):
    wid = _worker_id()
    nchunks = N // CA
    nmine = (nchunks - wid + NW - 1) // NW

    def chunk_body(i, carry):
        ck = wid + i * NW
        base = ck * CA
        pltpu.sync_copy(nb_hbm.at[pl.ds(base, CA)], nb_v)
        g_cp = pltpu.async_copy(gfeat_hbm.at[nb_v], g_v, sem)
        pltpu.sync_copy(node_hbm.at[pl.ds(base, CA)], nf_v)
        g_cp.wait()

        def row_body(r, rcarry):
            for j in range(D // LANES):
                sl = pl.ds(j * LANES, LANES)
                g_v[r, sl] = g_v[r, sl] + nf_v[r, sl]
            return rcarry

        lax.fori_loop(0, CA, row_body, 0)
        pltpu.sync_copy(g_v, out_hbm.at[pl.ds(base, CA)])
        return carry

    lax.fori_loop(0, nmine, chunk_body, 0)


@functools.partial(
    pl.kernel,
    out_type=[
        jax.ShapeDtypeStruct((L1, D), jnp.float32),
        jax.ShapeDtypeStruct((L2, D), jnp.float32),
        jax.ShapeDtypeStruct((L3, D), jnp.float32),
        jax.ShapeDtypeStruct((L1, D), jnp.float32),
        jax.ShapeDtypeStruct((L2, D), jnp.float32),
        jax.ShapeDtypeStruct((L3, D), jnp.float32),
    ],
    mesh=_MESH,
    scratch_types=[
        pltpu.VMEM((40 * CB2,), jnp.int32),
        pltpu.VMEM((CB2, D), jnp.float32),
        pltpu.VMEM((CB2, D), jnp.float32),
        pltpu.VMEM((CB2, D), jnp.float32),
        pltpu.SemaphoreType.DMA,
        pltpu.SemaphoreType.DMA,
        pltpu.SemaphoreType.DMA,
        pltpu.SemaphoreType.DMA,
        pltpu.SemaphoreType.DMA,
        pltpu.SemaphoreType.DMA,
    ],
)
def _gather_call(
    tng0_hbm, edge_hbm, n1, n2, n3, e1, e2, e3,
    o1, o2, o3, o4, o5, o6,
    idx_all_v, buf0_v, buf1_v, buf2_v, gs0, ss0, gs1, ss1, gs2, ss2,
):
    wid = _worker_id()
    NBUF = 3

    def do_gather(table, idxr, outr, length, c):
        nchunks = length // c
        w_chunks = (nchunks + NW - 1) // NW
        s_w = (wid * nchunks) // NW
        e_w = ((wid + 1) * nchunks) // NW
        nmine = e_w - s_w

        pltpu.sync_copy(
            idxr.at[pl.ds(s_w * c, w_chunks * c)],
            idx_all_v.at[pl.ds(0, w_chunks * c)],
        )

        buf_of = (
            buf0_v.at[pl.ds(0, c)],
            buf1_v.at[pl.ds(0, c)],
            buf2_v.at[pl.ds(0, c)],
        )
        gs_of = (gs0, gs1, gs2)
        ss_of = (ss0, ss1, ss2)

        def base(j):
            return (s_w + j) * c

        def idx_of(j):
            return idx_all_v.at[pl.ds(j * c, c)]

        def step(j, a):
            b = (a - 1) % NBUF

            @pl.when(j >= NBUF)
            def _():
                pltpu.make_async_copy(
                    buf_of[a], outr.at[pl.ds(0, c)], ss_of[a]
                ).wait()

            pltpu.async_copy(table.at[idx_of(j)], buf_of[a], gs_of[a])
            pltpu.make_async_copy(
                table.at[idx_of(0)], buf_of[b], gs_of[b]
            ).wait()
            pltpu.async_copy(buf_of[b], outr.at[pl.ds(base(j - 1), c)], ss_of[b])

        pltpu.async_copy(table.at[idx_of(0)], buf_of[0], gs_of[0])

        def body(j, carry):
            ph = j % NBUF
            for a in range(NBUF):
                @pl.when(ph == a)
                def _(a=a):
                    step(j, a)
            return carry

        lax.fori_loop(1, nmine, body, 0)

        last = nmine - 1
        for a in range(NBUF):
            @pl.when((last % NBUF) == a)
            def _(a=a):
                pltpu.make_async_copy(
                    table.at[idx_of(0)], buf_of[a], gs_of[a]
                ).wait()
                pltpu.async_copy(
                    buf_of[a], outr.at[pl.ds(base(last), c)], ss_of[a]
                )
        for a in range(NBUF):
            pltpu.make_async_copy(buf_of[a], outr.at[pl.ds(0, c)], ss_of[a]).wait()

    do_gather(tng0_hbm, n1, o1, L1, CB1)
    do_gather(tng0_hbm, n2, o2, L2, CB2)
    do_gather(tng0_hbm, n3, o3, L3, CB2)
    do_gather(edge_hbm, e1, o4, L1, CB1)
    do_gather(edge_hbm, e2, o5, L2, CB2)
    do_gather(edge_hbm, e3, o6, L3, CB2)


def kernel(node_feat, edge_feat, global_feat, n_img1, n_img2, n_img3,
           e_img1, e_img2, e_img3, node_batch):
    tng0 = _tng0_call(node_feat, global_feat, node_batch)
    tng1, tng2, tng3, te1, te2, te3 = _gather_call(
        tng0, edge_feat, n_img1, n_img2, n_img3, e_img1, e_img2, e_img3
    )
    return (tng0, tng1, tng2, tng3, edge_feat, te1, te2, te3)


# speedup vs baseline: 1.0807x; 1.0807x over previous
import functools

import jax
import jax.numpy as jnp
from jax import lax
from jax.experimental import pallas as pl
from jax.experimental.pallas import tpu as pltpu
from jax.experimental.pallas import tpu_sc as plsc

N, E, D, B = 10000, 160000, 256, 64
L1, L2, L3 = 40000, 80000, 160000

NC, NS = 2, 16
NW = NC * NS
LANES = 16

CA = 80
CB1 = 64
CB2 = 128

_MESH = plsc.VectorSubcoreMesh(
    core_axis_name="c", subcore_axis_name="s", num_cores=NC, num_subcores=NS
)


def _worker_id():
    return lax.axis_index("s") * NC + lax.axis_index("c")


RA = 400
NBLK = N // RA


def _tng0_tc_body(nb_ref, nf_ref, gf_ref, out_ref):
    nb = nb_ref[0, 0, :]
    onehot = (nb[:, None] == lax.broadcasted_iota(jnp.int32, (RA, B), 1))
    onehot = onehot.astype(jnp.float32)
    out_ref[...] = nf_ref[...] + jnp.dot(
        onehot, gf_ref[...], preferred_element_type=jnp.float32
    )


_tng0_tc = pl.pallas_call(
    _tng0_tc_body,
    grid=(NBLK,),
    in_specs=[
        pl.BlockSpec((1, 1, RA), lambda i: (i, 0, 0)),
        pl.BlockSpec((RA, D), lambda i: (i, 0)),
        pl.BlockSpec((B, D), lambda i: (0, 0)),
    ],
    out_specs=pl.BlockSpec((RA, D), lambda i: (i, 0)),
    out_shape=jax.ShapeDtypeStruct((N, D), jnp.float32),
)


@functools.partial(
    pl.kernel,
    out_type=[
        jax.ShapeDtypeStruct((L1, D), jnp.float32),
        jax.ShapeDtypeStruct((L2, D), jnp.float32),
        jax.ShapeDtypeStruct((L3, D), jnp.float32),
        jax.ShapeDtypeStruct((L1, D), jnp.float32),
        jax.ShapeDtypeStruct((L2, D), jnp.float32),
        jax.ShapeDtypeStruct((L3, D), jnp.float32),
    ],
    mesh=_MESH,
    scratch_types=[
        pltpu.VMEM((40 * CB2,), jnp.int32),
        pltpu.VMEM((CB2, D), jnp.float32),
        pltpu.VMEM((CB2, D), jnp.float32),
        pltpu.VMEM((CB2, D), jnp.float32),
        pltpu.SemaphoreType.DMA,
        pltpu.SemaphoreType.DMA,
        pltpu.SemaphoreType.DMA,
        pltpu.SemaphoreType.DMA,
        pltpu.SemaphoreType.DMA,
        pltpu.SemaphoreType.DMA,
    ],
)
def _gather_call(
    tng0_hbm, edge_hbm, n1, n2, n3, e1, e2, e3,
    o1, o2, o3, o4, o5, o6,
    idx_all_v, buf0_v, buf1_v, buf2_v, gs0, ss0, gs1, ss1, gs2, ss2,
):
    wid = _worker_id()
    NBUF = 3

    def do_gather(table, idxr, outr, length, c):
        nchunks = length // c
        w_chunks = (nchunks + NW - 1) // NW
        s_w = (wid * nchunks) // NW
        e_w = ((wid + 1) * nchunks) // NW
        nmine = e_w - s_w

        pltpu.sync_copy(
            idxr.at[pl.ds(s_w * c, w_chunks * c)],
            idx_all_v.at[pl.ds(0, w_chunks * c)],
        )

        buf_of = (
            buf0_v.at[pl.ds(0, c)],
            buf1_v.at[pl.ds(0, c)],
            buf2_v.at[pl.ds(0, c)],
        )
        gs_of = (gs0, gs1, gs2)
        ss_of = (ss0, ss1, ss2)

        def base(j):
            return (s_w + j) * c

        def idx_of(j):
            return idx_all_v.at[pl.ds(j * c, c)]

        def step(j, a):
            b = (a - 1) % NBUF

            @pl.when(j >= NBUF)
            def _():
                pltpu.make_async_copy(
                    buf_of[a], outr.at[pl.ds(0, c)], ss_of[a]
                ).wait()

            pltpu.async_copy(table.at[idx_of(j)], buf_of[a], gs_of[a])
            pltpu.make_async_copy(
                table.at[idx_of(0)], buf_of[b], gs_of[b]
            ).wait()
            pltpu.async_copy(buf_of[b], outr.at[pl.ds(base(j - 1), c)], ss_of[b])

        pltpu.async_copy(table.at[idx_of(0)], buf_of[0], gs_of[0])

        def body(j, carry):
            ph = j % NBUF
            for a in range(NBUF):
                @pl.when(ph == a)
                def _(a=a):
                    step(j, a)
            return carry

        lax.fori_loop(1, nmine, body, 0)

        last = nmine - 1
        for a in range(NBUF):
            @pl.when((last % NBUF) == a)
            def _(a=a):
                pltpu.make_async_copy(
                    table.at[idx_of(0)], buf_of[a], gs_of[a]
                ).wait()
                pltpu.async_copy(
                    buf_of[a], outr.at[pl.ds(base(last), c)], ss_of[a]
                )
        for a in range(NBUF):
            pltpu.make_async_copy(buf_of[a], outr.at[pl.ds(0, c)], ss_of[a]).wait()

    do_gather(tng0_hbm, n1, o1, L1, CB1)
    do_gather(tng0_hbm, n2, o2, L2, CB2)
    do_gather(tng0_hbm, n3, o3, L3, CB2)
    do_gather(edge_hbm, e1, o4, L1, CB1)
    do_gather(edge_hbm, e2, o5, L2, CB2)
    do_gather(edge_hbm, e3, o6, L3, CB2)


def kernel(node_feat, edge_feat, global_feat, n_img1, n_img2, n_img3,
           e_img1, e_img2, e_img3, node_batch):
    tng0 = _tng0_tc(node_batch.reshape(NBLK, 1, RA), node_feat, global_feat)
    tng1, tng2, tng3, te1, te2, te3 = _gather_call(
        tng0, edge_feat, n_img1, n_img2, n_img3, e_img1, e_img2, e_img3
    )
    return (tng0, tng1, tng2, tng3, edge_feat, te1, te2, te3)
